# baseline (speedup 1.0000x reference)
import functools
import math

import jax
import jax.numpy as jnp
from jax import lax
from jax.experimental import pallas as pl
from jax.experimental.pallas import tpu as pltpu
from jax.experimental.pallas import tpu_sc as plsc

N = 10000
E = 320000
D = 128
A = 10
R = 8
H = 64
AVG_NUM_NEIGHBORS = 32.0

NUM_SC = 2
NUM_TILES = 16
NW = NUM_SC * NUM_TILES
CHUNK = 80
KSLICE = 5
E_SLICE = E // KSLICE
E_PER_TILE = E_SLICE // NW
N_CHUNKS = E_PER_TILE // CHUNK
NQ = 6
NIDX = 4
N_PAD = 10240
ROWS_PER_TILE = N_PAD // NUM_TILES


def _mlp_body(ef_ref, ea_ref, w1_ref, w2_ref, w3_ref, w4_ref, out_ref):
    h = jnp.dot(ef_ref[...], w1_ref[...], preferred_element_type=jnp.float32)
    h = h * (1.0 / math.sqrt(R))
    h = h * jax.nn.sigmoid(h)
    h = jnp.dot(h.astype(jnp.bfloat16), w2_ref[...],
                preferred_element_type=jnp.float32)
    h = h * jax.nn.sigmoid(h)
    h = jnp.dot(h.astype(jnp.bfloat16), w3_ref[...],
                preferred_element_type=jnp.float32)
    h = h * jax.nn.sigmoid(h)
    tp = jnp.dot(h.astype(jnp.bfloat16), w4_ref[...],
                 preferred_element_type=jnp.float32)
    tp = tp * ea_ref[...]
    ti = jax.lax.bitcast_convert_type(tp, jnp.int32)
    rb = jax.lax.shift_right_logical(ti, 16) & 1
    ti = ti + 32767 + rb
    bf = jax.lax.shift_right_logical(ti, 16)
    lo = bf[:, :D // 2]
    hi = bf[:, D // 2:]
    out_ref[...] = lo | (hi << 16)


def _edge_mlp(edge_feats, edge_attrs, W1, W2, W3, W4):
    ne = edge_feats.shape[0]
    W2 = (W2 * (1.0 / math.sqrt(H))).astype(jnp.bfloat16)
    W3 = (W3 * (1.0 / math.sqrt(H))).astype(jnp.bfloat16)
    W4 = W4 * (1.0 / math.sqrt(H))

    qcol = jnp.asarray(
        [32 * (w // 16) + w % 16 for w in range(D // 2)]
        + [32 * (w // 16) + 16 + w % 16 for w in range(D // 2)],
        dtype=jnp.int32)
    W4 = W4[:, qcol].astype(jnp.bfloat16)
    BE = 8000
    grid = ne // BE
    return pl.pallas_call(
        _mlp_body,
        grid=(grid,),
        in_specs=[
            pl.BlockSpec((BE, R), lambda i: (i, 0)),
            pl.BlockSpec((BE, 1), lambda i: (i, 0)),
            pl.BlockSpec((R, H), lambda i: (0, 0)),
            pl.BlockSpec((H, H), lambda i: (0, 0)),
            pl.BlockSpec((H, H), lambda i: (0, 0)),
            pl.BlockSpec((H, D), lambda i: (0, 0)),
        ],
        out_specs=pl.BlockSpec((BE, D // 2), lambda i: (i, 0)),
        out_shape=jax.ShapeDtypeStruct((ne, D // 2), jnp.int32),
    )(edge_feats, edge_attrs, W1, W2, W3, W4)


def _up_body(nf_ref, w_ref, out_ref):
    out_ref[...] = jnp.dot(
        nf_ref[...], w_ref[...], preferred_element_type=jnp.float32
    ) * (1.0 / math.sqrt(D))


def _linear_up(node_feats, W_up):
    return pl.pallas_call(
        _up_body,
        out_shape=jax.ShapeDtypeStruct((N, D), jnp.float32),
    )(node_feats, W_up)


def _sc_body(x_hbm, tp_hbm, snd_hbm, rcv_hbm, init_hbm, out_hbm,
             sndb, rcvb, xrows, tprows, acc,
             si, st0, st1, sg0, sg1, ss0, ss1):
    sem_tp = (st0, st1)
    sem_g = (sg0, sg1)
    sem_s = (ss0, ss1)
    c = lax.axis_index("c")
    s = lax.axis_index("s")
    wid = c * NUM_TILES + s

    base0 = wid * E_PER_TILE

    def start_idx_tp(t, ib, xb):
        base = base0 + t * CHUNK
        pltpu.async_copy(snd_hbm.at[pl.ds(base, CHUNK)], sndb.at[ib, 0], si)
        pltpu.async_copy(rcv_hbm.at[pl.ds(base, CHUNK)], rcvb.at[ib, 0], si)
        pltpu.async_copy(tp_hbm.at[pl.ds(base, CHUNK)], tprows.at[xb],
                         sem_tp[xb])

    def wait_idx(ib):
        pltpu.make_async_copy(snd_hbm.at[pl.ds(base0, CHUNK)],
                              sndb.at[ib, 0], si).wait()
        pltpu.make_async_copy(rcv_hbm.at[pl.ds(base0, CHUNK)],
                              rcvb.at[ib, 0], si).wait()

    def wait_tp(xb):
        pltpu.make_async_copy(tp_hbm.at[pl.ds(base0, CHUNK)],
                              tprows.at[xb], sem_tp[xb]).wait()

    def start_gather(ib, xb):
        pltpu.async_copy(x_hbm.at[sndb.at[ib, 0]], xrows.at[xb], sem_g[xb])

    def wait_gather(ib, xb):
        pltpu.make_async_copy(x_hbm.at[sndb.at[ib, 0]], xrows.at[xb],
                              sem_g[xb]).wait()

    def start_scatter(ib, xb):
        pltpu.async_copy(xrows.at[xb], acc.at[rcvb.at[ib, 0]], sem_s[xb],
                         add=True)

    def wait_scatter(ib, xb):
        pltpu.make_async_copy(xrows.at[xb], acc.at[rcvb.at[ib, 0]],
                              sem_s[xb]).wait()

    def multiply(xb):
        def mul_body(i, carry):
            for g in range(D // 32):
                w = tprows[xb, i, pl.ds(g * 16, 16)]
                ta = jax.lax.bitcast_convert_type(w << 16, jnp.float32)
                tb = jax.lax.bitcast_convert_type(
                    w & jnp.int32(-65536), jnp.float32)
                sl0 = pl.ds(g * 32, 16)
                sl1 = pl.ds(g * 32 + 16, 16)
                xrows[xb, i, sl0] = xrows[xb, i, sl0] * ta
                xrows[xb, i, sl1] = xrows[xb, i, sl1] * tb
            return carry
        lax.fori_loop(0, CHUNK, mul_body, 0)

    def chunk_step(t, ib, xb, skip_scatter_wait=False):
        in1 = (ib + 1) % NIDX
        xn = xb ^ 1
        start_idx_tp(t + 1, in1, xn)
        wait_tp(xb)
        wait_gather(ib, xb)
        multiply(xb)
        if not skip_scatter_wait:
            wait_scatter((ib - 1) % NIDX, xn)
        wait_idx(in1)
        start_gather(in1, xn)
        start_scatter(ib, xb)

    pltpu.sync_copy(init_hbm.at[c, pl.ds(s * ROWS_PER_TILE, ROWS_PER_TILE)],
                    acc.at[pl.ds(s * ROWS_PER_TILE, ROWS_PER_TILE)])
    plsc.subcore_barrier()
    start_idx_tp(0, 0, 0)
    wait_idx(0)
    start_gather(0, 0)

    chunk_step(0, 0, 0, skip_scatter_wait=True)
    chunk_step(1, 1, 1)
    chunk_step(2, 2, 0)
    chunk_step(3, 3, 1)

    def quad_body(q, carry):
        chunk_step(4 * q + 0, 0, 0)
        chunk_step(4 * q + 1, 1, 1)
        chunk_step(4 * q + 2, 2, 0)
        chunk_step(4 * q + 3, 3, 1)
        return carry

    lax.fori_loop(1, NQ, quad_body, 0)

    wait_tp(0)
    wait_gather(0, 0)
    multiply(0)
    wait_scatter(3, 1)
    start_scatter(0, 0)
    wait_scatter(0, 0)
    plsc.subcore_barrier()

    pltpu.sync_copy(acc.at[pl.ds(s * ROWS_PER_TILE, ROWS_PER_TILE)],
                    out_hbm.at[c, pl.ds(s * ROWS_PER_TILE, ROWS_PER_TILE)])


def _sc_scatter(x, tp_scaled, sender, receiver, init):
    mesh = plsc.VectorSubcoreMesh(core_axis_name="c", subcore_axis_name="s")
    f = functools.partial(
        pl.kernel,
        out_type=jax.ShapeDtypeStruct((NUM_SC, N_PAD, D), jnp.float32),
        mesh=mesh,
        scratch_types=[
            pltpu.VMEM((NIDX, 1, CHUNK), jnp.int32),
            pltpu.VMEM((NIDX, 1, CHUNK), jnp.int32),
            pltpu.VMEM((2, CHUNK, D), jnp.float32),
            pltpu.VMEM((2, CHUNK, D // 2), jnp.int32),
            pltpu.VMEM_SHARED((N_PAD, D), jnp.float32),
            pltpu.SemaphoreType.DMA,
            pltpu.SemaphoreType.DMA,
            pltpu.SemaphoreType.DMA,
            pltpu.SemaphoreType.DMA,
            pltpu.SemaphoreType.DMA,
            pltpu.SemaphoreType.DMA,
            pltpu.SemaphoreType.DMA,
        ],
    )(_sc_body)
    return f(x, tp_scaled, sender, receiver, init)


def _final_body(parts_ref, na_ref, wlin_ref, wskip_ref, out_ref):
    m = jnp.zeros(out_ref.shape, jnp.float32)
    for k in range(KSLICE):
        m = m + parts_ref[k, 0] + parts_ref[k, 1]
    z = jnp.dot(m, wlin_ref[...], preferred_element_type=jnp.float32)
    z = z * (1.0 / (math.sqrt(D) * AVG_NUM_NEIGHBORS))
    acc = jnp.zeros(out_ref.shape, jnp.float32)
    for v in range(A):
        acc = acc + jnp.dot(
            z, wskip_ref[:, v, :], preferred_element_type=jnp.float32
        ) * na_ref[:, v:v + 1]
    out_ref[...] = acc * (1.0 / math.sqrt(D * A))


def _final(parts, node_attrs, W_lin, W_skip):
    BN = 2000
    grid = N // BN
    return pl.pallas_call(
        _final_body,
        grid=(grid,),
        in_specs=[
            pl.BlockSpec((KSLICE, NUM_SC, BN, D), lambda i: (0, 0, i, 0)),
            pl.BlockSpec((BN, A), lambda i: (i, 0)),
            pl.BlockSpec((D, D), lambda i: (0, 0)),
            pl.BlockSpec((D, A, D), lambda i: (0, 0, 0)),
        ],
        out_specs=pl.BlockSpec((BN, D), lambda i: (i, 0)),
        out_shape=jax.ShapeDtypeStruct((N, D), jnp.float32),
    )(parts, node_attrs, W_lin, W_skip)


def kernel(node_attrs, node_feats, edge_attrs, edge_feats, edge_index,
           W_up, W1, W2, W3, W4, W_lin, W_skip):
    edge_index = edge_index.astype(jnp.int32)
    x = _linear_up(node_feats, W_up)
    zeros = jnp.zeros((NUM_SC, N_PAD, D), jnp.float32)
    parts = []
    for k in range(KSLICE):
        sl = slice(k * E_SLICE, (k + 1) * E_SLICE)
        tp_k = _edge_mlp(edge_feats[sl], edge_attrs[sl], W1, W2, W3, W4)
        parts.append(_sc_scatter(x, tp_k, edge_index[0, sl],
                                 edge_index[1, sl], zeros))
    return _final(jnp.stack(parts), node_attrs, W_lin, W_skip)


# speedup vs baseline: 1.0331x; 1.0331x over previous
import functools
import math

import jax
import jax.numpy as jnp
from jax import lax
from jax.experimental import pallas as pl
from jax.experimental.pallas import tpu as pltpu
from jax.experimental.pallas import tpu_sc as plsc

N = 10000
E = 320000
D = 128
A = 10
R = 8
H = 64
AVG_NUM_NEIGHBORS = 32.0

NUM_SC = 2
NUM_TILES = 16
NW = NUM_SC * NUM_TILES
CHUNK = 80
KSLICE = 5
E_SLICE = E // KSLICE
E_PER_TILE = E_SLICE // NW
N_CHUNKS = E_PER_TILE // CHUNK
NQ = 6
NIDX = 4
N_PAD = 10240
ROWS_PER_TILE = N_PAD // NUM_TILES


def _mlp_body(ef_ref, ea_ref, w1_ref, w2_ref, w3_ref, w4_ref, out_ref):
    h = jnp.dot(ef_ref[...], w1_ref[...], preferred_element_type=jnp.float32)
    h = h * (1.0 / math.sqrt(R))
    h = h * jax.nn.sigmoid(h)
    h = jnp.dot(h.astype(jnp.bfloat16), w2_ref[...],
                preferred_element_type=jnp.float32)
    h = h * jax.nn.sigmoid(h)
    h = jnp.dot(h.astype(jnp.bfloat16), w3_ref[...],
                preferred_element_type=jnp.float32)
    h = h * jax.nn.sigmoid(h)
    tp = jnp.dot(h.astype(jnp.bfloat16), w4_ref[...],
                 preferred_element_type=jnp.float32)
    tp = tp * ea_ref[...]
    ti = jax.lax.bitcast_convert_type(tp, jnp.int32)
    rb = jax.lax.shift_right_logical(ti, 16) & 1
    ti = ti + 32767 + rb
    bf = jax.lax.shift_right_logical(ti, 16)
    lo = bf[:, :D // 2]
    hi = bf[:, D // 2:]
    out_ref[...] = lo | (hi << 16)


def _edge_mlp(edge_feats, edge_attrs, W1, W2, W3, W4):
    ne = edge_feats.shape[0]
    W2 = (W2 * (1.0 / math.sqrt(H))).astype(jnp.bfloat16)
    W3 = (W3 * (1.0 / math.sqrt(H))).astype(jnp.bfloat16)
    W4 = W4 * (1.0 / math.sqrt(H))

    qcol = jnp.asarray(
        [32 * (w // 16) + w % 16 for w in range(D // 2)]
        + [32 * (w // 16) + 16 + w % 16 for w in range(D // 2)],
        dtype=jnp.int32)
    W4 = W4[:, qcol].astype(jnp.bfloat16)
    BE = 8000
    grid = ne // BE
    return pl.pallas_call(
        _mlp_body,
        grid=(grid,),
        in_specs=[
            pl.BlockSpec((BE, R), lambda i: (i, 0)),
            pl.BlockSpec((BE, 1), lambda i: (i, 0)),
            pl.BlockSpec((R, H), lambda i: (0, 0)),
            pl.BlockSpec((H, H), lambda i: (0, 0)),
            pl.BlockSpec((H, H), lambda i: (0, 0)),
            pl.BlockSpec((H, D), lambda i: (0, 0)),
        ],
        out_specs=pl.BlockSpec((BE, D // 2), lambda i: (i, 0)),
        out_shape=jax.ShapeDtypeStruct((ne, D // 2), jnp.int32),
    )(edge_feats, edge_attrs, W1, W2, W3, W4)


def _up_body(nf_ref, w_ref, out_ref):
    out_ref[...] = jnp.dot(
        nf_ref[...], w_ref[...], preferred_element_type=jnp.float32
    ) * (1.0 / math.sqrt(D))


def _linear_up(node_feats, W_up):
    return pl.pallas_call(
        _up_body,
        out_shape=jax.ShapeDtypeStruct((N, D), jnp.float32),
    )(node_feats, W_up)


def _sc_body(x_hbm, tp_hbm, snd_hbm, rcv_hbm, init_hbm, out_hbm,
             sndb, rcvb, xrows, tprows, acc,
             si, st0, st1, sg0, sg1, ss0, ss1):
    sem_tp = (st0, st1)
    sem_g = (sg0, sg1)
    sem_s = (ss0, ss1)
    c = lax.axis_index("c")
    s = lax.axis_index("s")
    wid = c * NUM_TILES + s

    base0 = wid * E_PER_TILE

    def start_idx_tp(t, ib, xb):
        base = base0 + t * CHUNK
        pltpu.async_copy(snd_hbm.at[pl.ds(base, CHUNK)], sndb.at[ib, 0], si)
        pltpu.async_copy(rcv_hbm.at[pl.ds(base, CHUNK)], rcvb.at[ib, 0], si)
        pltpu.async_copy(tp_hbm.at[pl.ds(base, CHUNK)], tprows.at[xb],
                         sem_tp[xb])

    def wait_idx(ib):
        pltpu.make_async_copy(snd_hbm.at[pl.ds(base0, CHUNK)],
                              sndb.at[ib, 0], si).wait()
        pltpu.make_async_copy(rcv_hbm.at[pl.ds(base0, CHUNK)],
                              rcvb.at[ib, 0], si).wait()

    def wait_tp(xb):
        pltpu.make_async_copy(tp_hbm.at[pl.ds(base0, CHUNK)],
                              tprows.at[xb], sem_tp[xb]).wait()

    def start_gather(ib, xb):
        pltpu.async_copy(x_hbm.at[sndb.at[ib, 0]], xrows.at[xb], sem_g[xb])

    def wait_gather(ib, xb):
        pltpu.make_async_copy(x_hbm.at[sndb.at[ib, 0]], xrows.at[xb],
                              sem_g[xb]).wait()

    def start_scatter(ib, xb):
        pltpu.async_copy(xrows.at[xb], acc.at[rcvb.at[ib, 0]], sem_s[xb],
                         add=True)

    def wait_scatter(ib, xb):
        pltpu.make_async_copy(xrows.at[xb], acc.at[rcvb.at[ib, 0]],
                              sem_s[xb]).wait()

    def multiply(xb):
        @plsc.parallel_loop(0, CHUNK, unroll=4)
        def mul_body(i):
            for g in range(D // 32):
                w = tprows[xb, i, pl.ds(g * 16, 16)]
                ta = jax.lax.bitcast_convert_type(w << 16, jnp.float32)
                tb = jax.lax.bitcast_convert_type(
                    w & jnp.int32(-65536), jnp.float32)
                sl0 = pl.ds(g * 32, 16)
                sl1 = pl.ds(g * 32 + 16, 16)
                xrows[xb, i, sl0] = xrows[xb, i, sl0] * ta
                xrows[xb, i, sl1] = xrows[xb, i, sl1] * tb

    def chunk_step(t, ib, xb, skip_scatter_wait=False):
        in1 = (ib + 1) % NIDX
        xn = xb ^ 1
        start_idx_tp(t + 1, in1, xn)
        wait_tp(xb)
        wait_gather(ib, xb)
        multiply(xb)
        if not skip_scatter_wait:
            wait_scatter((ib - 1) % NIDX, xn)
        wait_idx(in1)
        start_gather(in1, xn)
        start_scatter(ib, xb)

    pltpu.sync_copy(init_hbm.at[c, pl.ds(s * ROWS_PER_TILE, ROWS_PER_TILE)],
                    acc.at[pl.ds(s * ROWS_PER_TILE, ROWS_PER_TILE)])
    plsc.subcore_barrier()
    start_idx_tp(0, 0, 0)
    wait_idx(0)
    start_gather(0, 0)

    chunk_step(0, 0, 0, skip_scatter_wait=True)
    chunk_step(1, 1, 1)
    chunk_step(2, 2, 0)
    chunk_step(3, 3, 1)

    def quad_body(q, carry):
        chunk_step(4 * q + 0, 0, 0)
        chunk_step(4 * q + 1, 1, 1)
        chunk_step(4 * q + 2, 2, 0)
        chunk_step(4 * q + 3, 3, 1)
        return carry

    lax.fori_loop(1, NQ, quad_body, 0)

    wait_tp(0)
    wait_gather(0, 0)
    multiply(0)
    wait_scatter(3, 1)
    start_scatter(0, 0)
    wait_scatter(0, 0)
    plsc.subcore_barrier()

    pltpu.sync_copy(acc.at[pl.ds(s * ROWS_PER_TILE, ROWS_PER_TILE)],
                    out_hbm.at[c, pl.ds(s * ROWS_PER_TILE, ROWS_PER_TILE)])


def _sc_scatter(x, tp_scaled, sender, receiver, init):
    mesh = plsc.VectorSubcoreMesh(core_axis_name="c", subcore_axis_name="s")
    f = functools.partial(
        pl.kernel,
        out_type=jax.ShapeDtypeStruct((NUM_SC, N_PAD, D), jnp.float32),
        mesh=mesh,
        scratch_types=[
            pltpu.VMEM((NIDX, 1, CHUNK), jnp.int32),
            pltpu.VMEM((NIDX, 1, CHUNK), jnp.int32),
            pltpu.VMEM((2, CHUNK, D), jnp.float32),
            pltpu.VMEM((2, CHUNK, D // 2), jnp.int32),
            pltpu.VMEM_SHARED((N_PAD, D), jnp.float32),
            pltpu.SemaphoreType.DMA,
            pltpu.SemaphoreType.DMA,
            pltpu.SemaphoreType.DMA,
            pltpu.SemaphoreType.DMA,
            pltpu.SemaphoreType.DMA,
            pltpu.SemaphoreType.DMA,
            pltpu.SemaphoreType.DMA,
        ],
    )(_sc_body)
    return f(x, tp_scaled, sender, receiver, init)


def _final_body(parts_ref, na_ref, wlin_ref, wskip_ref, out_ref):
    m = parts_ref[0] + parts_ref[1]
    z = jnp.dot(m, wlin_ref[...], preferred_element_type=jnp.float32)
    z = z * (1.0 / (math.sqrt(D) * AVG_NUM_NEIGHBORS))
    acc = jnp.zeros(out_ref.shape, jnp.float32)
    for v in range(A):
        acc = acc + jnp.dot(
            z, wskip_ref[:, v, :], preferred_element_type=jnp.float32
        ) * na_ref[:, v:v + 1]
    out_ref[...] = acc * (1.0 / math.sqrt(D * A))


def _final(parts, node_attrs, W_lin, W_skip):
    BN = 2000
    grid = N // BN
    return pl.pallas_call(
        _final_body,
        grid=(grid,),
        in_specs=[
            pl.BlockSpec((NUM_SC, BN, D), lambda i: (0, i, 0)),
            pl.BlockSpec((BN, A), lambda i: (i, 0)),
            pl.BlockSpec((D, D), lambda i: (0, 0)),
            pl.BlockSpec((D, A, D), lambda i: (0, 0, 0)),
        ],
        out_specs=pl.BlockSpec((BN, D), lambda i: (i, 0)),
        out_shape=jax.ShapeDtypeStruct((N, D), jnp.float32),
    )(parts, node_attrs, W_lin, W_skip)


def kernel(node_attrs, node_feats, edge_attrs, edge_feats, edge_index,
           W_up, W1, W2, W3, W4, W_lin, W_skip):
    edge_index = edge_index.astype(jnp.int32)
    x = _linear_up(node_feats, W_up)
    parts = jnp.zeros((NUM_SC, N_PAD, D), jnp.float32)
    for k in range(KSLICE):
        sl = slice(k * E_SLICE, (k + 1) * E_SLICE)
        tp_k = _edge_mlp(edge_feats[sl], edge_attrs[sl], W1, W2, W3, W4)
        parts = _sc_scatter(x, tp_k, edge_index[0, sl], edge_index[1, sl],
                            parts)
    return _final(parts, node_attrs, W_lin, W_skip)


# speedup vs baseline: 1.0386x; 1.0054x over previous
import functools
import math

import jax
import jax.numpy as jnp
from jax import lax
from jax.experimental import pallas as pl
from jax.experimental.pallas import tpu as pltpu
from jax.experimental.pallas import tpu_sc as plsc

N = 10000
E = 320000
D = 128
A = 10
R = 8
H = 64
AVG_NUM_NEIGHBORS = 32.0

NUM_SC = 2
NUM_TILES = 16
NW = NUM_SC * NUM_TILES
CHUNK = 80
KSLICE = 5
E_SLICE = E // KSLICE
E_PER_TILE = E_SLICE // NW
N_CHUNKS = E_PER_TILE // CHUNK
NQ = 6
NIDX = 4
N_PAD = 10240
ROWS_PER_TILE = N_PAD // NUM_TILES


def _mlp_body(ef_ref, ea_ref, w1_ref, w2_ref, w3_ref, w4_ref, out_ref):
    h = jnp.dot(ef_ref[...], w1_ref[...], preferred_element_type=jnp.float32)
    h = h * (1.0 / math.sqrt(R))
    h = h * jax.nn.sigmoid(h)
    h = jnp.dot(h.astype(jnp.bfloat16), w2_ref[...],
                preferred_element_type=jnp.float32)
    h = h * jax.nn.sigmoid(h)
    h = jnp.dot(h.astype(jnp.bfloat16), w3_ref[...],
                preferred_element_type=jnp.float32)
    h = h * jax.nn.sigmoid(h)
    tp = jnp.dot(h.astype(jnp.bfloat16), w4_ref[...],
                 preferred_element_type=jnp.float32)
    tp = tp * ea_ref[...]
    ti = jax.lax.bitcast_convert_type(tp, jnp.int32)
    rb = jax.lax.shift_right_logical(ti, 16) & 1
    ti = ti + 32767 + rb
    bf = jax.lax.shift_right_logical(ti, 16)
    lo = bf[:, :D // 2]
    hi = bf[:, D // 2:]
    out_ref[...] = lo | (hi << 16)


def _edge_mlp(edge_feats, edge_attrs, W1, W2, W3, W4):
    ne = edge_feats.shape[0]
    W2 = (W2 * (1.0 / math.sqrt(H))).astype(jnp.bfloat16)
    W3 = (W3 * (1.0 / math.sqrt(H))).astype(jnp.bfloat16)
    W4 = W4 * (1.0 / math.sqrt(H))

    qcol = jnp.asarray(
        [32 * (w // 16) + w % 16 for w in range(D // 2)]
        + [32 * (w // 16) + 16 + w % 16 for w in range(D // 2)],
        dtype=jnp.int32)
    W4 = W4[:, qcol].astype(jnp.bfloat16)
    BE = 8000
    grid = ne // BE
    return pl.pallas_call(
        _mlp_body,
        grid=(grid,),
        in_specs=[
            pl.BlockSpec((BE, R), lambda i: (i, 0)),
            pl.BlockSpec((BE, 1), lambda i: (i, 0)),
            pl.BlockSpec((R, H), lambda i: (0, 0)),
            pl.BlockSpec((H, H), lambda i: (0, 0)),
            pl.BlockSpec((H, H), lambda i: (0, 0)),
            pl.BlockSpec((H, D), lambda i: (0, 0)),
        ],
        out_specs=pl.BlockSpec((BE, D // 2), lambda i: (i, 0)),
        out_shape=jax.ShapeDtypeStruct((ne, D // 2), jnp.int32),
    )(edge_feats, edge_attrs, W1, W2, W3, W4)


def _up_body(nf_ref, w_ref, out_ref):
    out_ref[...] = jnp.dot(
        nf_ref[...], w_ref[...], preferred_element_type=jnp.float32
    ) * (1.0 / math.sqrt(D))


def _linear_up(node_feats, W_up):
    return pl.pallas_call(
        _up_body,
        out_shape=jax.ShapeDtypeStruct((N, D), jnp.float32),
    )(node_feats, W_up)


def _sc_body(x_hbm, tp_hbm, snd_hbm, rcv_hbm, init_hbm, out_hbm,
             sndb, rcvb, xrows, tprows, acc,
             si, st0, st1, sg0, sg1, ss0, ss1):
    sem_tp = (st0, st1)
    sem_g = (sg0, sg1)
    sem_s = (ss0, ss1)
    c = lax.axis_index("c")
    s = lax.axis_index("s")
    wid = c * NUM_TILES + s

    base0 = wid * E_PER_TILE

    def start_idx_tp(t, ib, xb):
        base = base0 + t * CHUNK
        pltpu.async_copy(snd_hbm.at[pl.ds(base, CHUNK)], sndb.at[ib, 0], si)
        pltpu.async_copy(rcv_hbm.at[pl.ds(base, CHUNK)], rcvb.at[ib, 0], si)
        pltpu.async_copy(tp_hbm.at[pl.ds(base, CHUNK)], tprows.at[xb],
                         sem_tp[xb])

    def wait_idx(ib):
        pltpu.make_async_copy(snd_hbm.at[pl.ds(base0, CHUNK)],
                              sndb.at[ib, 0], si).wait()
        pltpu.make_async_copy(rcv_hbm.at[pl.ds(base0, CHUNK)],
                              rcvb.at[ib, 0], si).wait()

    def wait_tp(xb):
        pltpu.make_async_copy(tp_hbm.at[pl.ds(base0, CHUNK)],
                              tprows.at[xb], sem_tp[xb]).wait()

    def start_gather(ib, xb):
        pltpu.async_copy(x_hbm.at[sndb.at[ib, 0]], xrows.at[xb], sem_g[xb])

    def wait_gather(ib, xb):
        pltpu.make_async_copy(x_hbm.at[sndb.at[ib, 0]], xrows.at[xb],
                              sem_g[xb]).wait()

    def start_scatter(ib, xb):
        pltpu.async_copy(xrows.at[xb], acc.at[rcvb.at[ib, 0]], sem_s[xb],
                         add=True)

    def wait_scatter(ib, xb):
        pltpu.make_async_copy(xrows.at[xb], acc.at[rcvb.at[ib, 0]],
                              sem_s[xb]).wait()

    def multiply(xb):
        def mul_body(i, carry):
            for g in range(D // 32):
                w = tprows[xb, i, pl.ds(g * 16, 16)]
                ta = jax.lax.bitcast_convert_type(w << 16, jnp.float32)
                tb = jax.lax.bitcast_convert_type(
                    w & jnp.int32(-65536), jnp.float32)
                sl0 = pl.ds(g * 32, 16)
                sl1 = pl.ds(g * 32 + 16, 16)
                xrows[xb, i, sl0] = xrows[xb, i, sl0] * ta
                xrows[xb, i, sl1] = xrows[xb, i, sl1] * tb
            return carry
        lax.fori_loop(0, CHUNK, mul_body, 0)

    def chunk_step(t, ib, xb, skip_scatter_wait=False):
        in1 = (ib + 1) % NIDX
        xn = xb ^ 1
        start_idx_tp(t + 1, in1, xn)
        wait_tp(xb)
        wait_gather(ib, xb)
        multiply(xb)
        if not skip_scatter_wait:
            wait_scatter((ib - 1) % NIDX, xn)
        wait_idx(in1)
        start_gather(in1, xn)
        start_scatter(ib, xb)

    pltpu.sync_copy(init_hbm.at[c, pl.ds(s * ROWS_PER_TILE, ROWS_PER_TILE)],
                    acc.at[pl.ds(s * ROWS_PER_TILE, ROWS_PER_TILE)])
    plsc.subcore_barrier()
    start_idx_tp(0, 0, 0)
    wait_idx(0)
    start_gather(0, 0)

    chunk_step(0, 0, 0, skip_scatter_wait=True)
    chunk_step(1, 1, 1)
    chunk_step(2, 2, 0)
    chunk_step(3, 3, 1)

    def quad_body(q, carry):
        chunk_step(4 * q + 0, 0, 0)
        chunk_step(4 * q + 1, 1, 1)
        chunk_step(4 * q + 2, 2, 0)
        chunk_step(4 * q + 3, 3, 1)
        return carry

    lax.fori_loop(1, NQ, quad_body, 0)

    wait_tp(0)
    wait_gather(0, 0)
    multiply(0)
    wait_scatter(3, 1)
    start_scatter(0, 0)
    wait_scatter(0, 0)
    plsc.subcore_barrier()

    pltpu.sync_copy(acc.at[pl.ds(s * ROWS_PER_TILE, ROWS_PER_TILE)],
                    out_hbm.at[c, pl.ds(s * ROWS_PER_TILE, ROWS_PER_TILE)])


def _sc_scatter(x, tp_scaled, sender, receiver, init):
    mesh = plsc.VectorSubcoreMesh(core_axis_name="c", subcore_axis_name="s")
    f = functools.partial(
        pl.kernel,
        out_type=jax.ShapeDtypeStruct((NUM_SC, N_PAD, D), jnp.float32),
        mesh=mesh,
        scratch_types=[
            pltpu.VMEM((NIDX, 1, CHUNK), jnp.int32),
            pltpu.VMEM((NIDX, 1, CHUNK), jnp.int32),
            pltpu.VMEM((2, CHUNK, D), jnp.float32),
            pltpu.VMEM((2, CHUNK, D // 2), jnp.int32),
            pltpu.VMEM_SHARED((N_PAD, D), jnp.float32),
            pltpu.SemaphoreType.DMA,
            pltpu.SemaphoreType.DMA,
            pltpu.SemaphoreType.DMA,
            pltpu.SemaphoreType.DMA,
            pltpu.SemaphoreType.DMA,
            pltpu.SemaphoreType.DMA,
            pltpu.SemaphoreType.DMA,
        ],
    )(_sc_body)
    return f(x, tp_scaled, sender, receiver, init)


def _final_body(parts_ref, na_ref, wlin_ref, wskip_ref, out_ref):
    m = parts_ref[0] + parts_ref[1]
    z = jnp.dot(m, wlin_ref[...], preferred_element_type=jnp.float32)
    z = z * (1.0 / (math.sqrt(D) * AVG_NUM_NEIGHBORS))
    acc = jnp.zeros(out_ref.shape, jnp.float32)
    for v in range(A):
        acc = acc + jnp.dot(
            z, wskip_ref[:, v, :], preferred_element_type=jnp.float32
        ) * na_ref[:, v:v + 1]
    out_ref[...] = acc * (1.0 / math.sqrt(D * A))


def _final(parts, node_attrs, W_lin, W_skip):
    BN = 2000
    grid = N // BN
    return pl.pallas_call(
        _final_body,
        grid=(grid,),
        in_specs=[
            pl.BlockSpec((NUM_SC, BN, D), lambda i: (0, i, 0)),
            pl.BlockSpec((BN, A), lambda i: (i, 0)),
            pl.BlockSpec((D, D), lambda i: (0, 0)),
            pl.BlockSpec((D, A, D), lambda i: (0, 0, 0)),
        ],
        out_specs=pl.BlockSpec((BN, D), lambda i: (i, 0)),
        out_shape=jax.ShapeDtypeStruct((N, D), jnp.float32),
    )(parts, node_attrs, W_lin, W_skip)


def kernel(node_attrs, node_feats, edge_attrs, edge_feats, edge_index,
           W_up, W1, W2, W3, W4, W_lin, W_skip):
    edge_index = edge_index.astype(jnp.int32)
    x = _linear_up(node_feats, W_up)
    parts = jnp.zeros((NUM_SC, N_PAD, D), jnp.float32)
    for k in range(KSLICE):
        sl = slice(k * E_SLICE, (k + 1) * E_SLICE)
        tp_k = _edge_mlp(edge_feats[sl], edge_attrs[sl], W1, W2, W3, W4)
        parts = _sc_scatter(x, tp_k, edge_index[0, sl], edge_index[1, sl],
                            parts)
    return _final(parts, node_attrs, W_lin, W_skip)
